# baseline (speedup 1.0000x reference)
import jax
import jax.numpy as jnp
from jax import lax
from jax.experimental import pallas as pl
from jax.experimental.pallas import tpu as pltpu
from jax.experimental.pallas import tpu_sc as plsc

N = 10000
DEG = 16
D = 256
KK = 8
NC = 2
NS = 16
NW = NC * NS
CH = 8
CE = CH * DEG
NCHUNK = N // CH
CBASE = NCHUNK // NW
CEXTRA = NCHUNK % NW
TPW = CBASE + 1
DC = D // 16
TCB = 1000

_f32 = jnp.float32
_i32 = jnp.int32


def _x1_body(wfc, bfc, x, x1):
    x1[...] = jnp.dot(x[...], wfc[...], preferred_element_type=_f32) + bfc[...]


def _tc_x1(X, W_fc, bfc_row):
    return pl.pallas_call(
        _x1_body,
        grid=(N // TCB,),
        in_specs=[
            pl.BlockSpec((D, D), lambda i: (0, 0)),
            pl.BlockSpec((1, D), lambda i: (0, 0)),
            pl.BlockSpec((TCB, D), lambda i: (i, 0)),
        ],
        out_specs=pl.BlockSpec((TCB, D), lambda i: (i, 0)),
        out_shape=jax.ShapeDtypeStruct((N, D), _f32),
    )(W_fc, bfc_row, X)


def _x2y_body(agg, x, wbil, y):
    h = agg[...] + x[...]
    n = jnp.sqrt(jnp.sum(h * h, axis=1, keepdims=True))
    x2 = h / jnp.maximum(n, 1e-12)
    y[...] = jnp.dot(x2, wbil[...], preferred_element_type=_f32)


def _tc_x2y(agg, X, W_bil):
    return pl.pallas_call(
        _x2y_body,
        grid=(N // TCB,),
        in_specs=[
            pl.BlockSpec((TCB, D), lambda i: (i, 0)),
            pl.BlockSpec((TCB, D), lambda i: (i, 0)),
            pl.BlockSpec((D, D), lambda i: (0, 0)),
        ],
        out_specs=pl.BlockSpec((TCB, D), lambda i: (i, 0)),
        out_shape=jax.ShapeDtypeStruct((N, D), _f32),
    )(agg, X, W_bil)


def _worker_extent(wid):
    cnt = CBASE + (wid < CEXTRA).astype(_i32)
    base = CBASE * wid + jnp.minimum(wid, CEXTRA)
    return base, cnt


def _stage_edges(src_hbm, dst_v, e0w):
    pltpu.sync_copy(src_hbm.at[pl.ds(e0w, CBASE * CE)], dst_v.at[pl.ds(0, CBASE * CE)])


def _stage_edges_tail(src_hbm, dst_v, e0w, cnt):
    @pl.when(cnt > CBASE)
    def _():
        pltpu.sync_copy(
            src_hbm.at[pl.ds(e0w + CBASE * CE, CE)],
            dst_v.at[pl.ds(CBASE * CE, CE)],
        )


def _sc_agg_body(
    x_hbm, cols_hbm, vals_hbm, agg_hbm,
    idxv, vvv, xg0, xg1, ab0, ab1, sem0, sem1, semo0, semo1,
):
    wid = lax.axis_index("s") * NC + lax.axis_index("c")
    base, cnt = _worker_extent(wid)
    e0w = base * CE
    _stage_edges(cols_hbm, idxv, e0w)
    _stage_edges(vals_hbm, vvv, e0w)
    _stage_edges_tail(cols_hbm, idxv, e0w, cnt)
    _stage_edges_tail(vals_hbm, vvv, e0w, cnt)
    xg = (xg0, xg1)
    ab = (ab0, ab1)
    sem = (sem0, sem1)
    semo = (semo0, semo1)

    def gather(t, b):
        return pltpu.make_async_copy(
            x_hbm.at[idxv.at[pl.ds(t * CE, CE)]], xg[b], sem[b]
        )

    def issue(t, b):
        @pl.when(t < cnt)
        def _():
            pltpu.async_copy(x_hbm.at[idxv.at[pl.ds(t * CE, CE)]], xg[b], sem[b])

    def work(t, b):
        issue(t + 1, 1 - b)

        @pl.when(t < cnt)
        def _():
            gather(t, b).wait()

            @pl.when(t >= 2)
            def _():
                pltpu.make_async_copy(
                    ab[b], agg_hbm.at[pl.ds((base + t - 2) * CH, CH)], semo[b]
                ).wait()

            xg_v = xg[b]
            ab_v = ab[b]

            def row(r, _):
                eb = t * CE + r * DEG
                accs = [jnp.zeros((16,), _f32) for _ in range(DC)]
                for l in range(DEG):
                    w = plsc.load_gather(vvv, [jnp.full((16,), eb + l, _i32)])
                    eg = r * DEG + l
                    for c in range(DC):
                        accs[c] = accs[c] + w * xg_v[eg, pl.ds(16 * c, 16)]
                for c in range(DC):
                    ab_v[r, pl.ds(16 * c, 16)] = accs[c]
                return 0

            lax.fori_loop(0, CH, row, 0, unroll=2)
            pltpu.async_copy(ab_v, agg_hbm.at[pl.ds((base + t) * CH, CH)], semo[b])

    issue(0, 0)

    def outer(tt, _):
        work(2 * tt, 0)
        work(2 * tt + 1, 1)
        return 0

    lax.fori_loop(0, TPW // 2, outer, 0)

    tl = cnt - 1
    for b in range(2):
        tb = tl - ((tl - b) % 2)
        pltpu.make_async_copy(
            ab[b], agg_hbm.at[pl.ds((base + tb) * CH, CH)], semo[b]
        ).wait()


def _sc_agg(X, cols, vals):
    return pl.kernel(
        _sc_agg_body,
        out_type=jax.ShapeDtypeStruct((N, D), _f32),
        mesh=plsc.VectorSubcoreMesh(core_axis_name="c", subcore_axis_name="s"),
        compiler_params=pltpu.CompilerParams(needs_layout_passes=False),
        scratch_types=[
            pltpu.VMEM((TPW * CE,), _i32),
            pltpu.VMEM((TPW * CE,), _f32),
            pltpu.VMEM((CE, D), _f32),
            pltpu.VMEM((CE, D), _f32),
            pltpu.VMEM((CH, D), _f32),
            pltpu.VMEM((CH, D), _f32),
            pltpu.SemaphoreType.DMA,
            pltpu.SemaphoreType.DMA,
            pltpu.SemaphoreType.DMA,
            pltpu.SemaphoreType.DMA,
        ],
    )(X, cols, vals)


def _sc_score_body(
    x1_hbm, y_hbm, cols_hbm, bc_hbm,
    vals_hbm, sel_hbm,
    idxv, qg0, qg1, y0, y1, bc_v, pb_v,
    vb0, vb1, cb0, cb1, sem0, sem1, semy0, semy1, semo0, semo1,
):
    wid = lax.axis_index("s") * NC + lax.axis_index("c")
    base, cnt = _worker_extent(wid)
    e0w = base * CE
    pltpu.sync_copy(bc_hbm, bc_v)
    _stage_edges(cols_hbm, idxv, e0w)
    _stage_edges_tail(cols_hbm, idxv, e0w, cnt)
    qg = (qg0, qg1)
    yv = (y0, y1)
    vb = (vb0, vb1)
    cb = (cb0, cb1)
    sem = (sem0, sem1)
    semy = (semy0, semy1)
    semo = (semo0, semo1)

    def issue(t, b):
        @pl.when(t < cnt)
        def _():
            pltpu.async_copy(x1_hbm.at[idxv.at[pl.ds(t * CE, CE)]], qg[b], sem[b])
            pltpu.async_copy(
                y_hbm.at[pl.ds((base + t) * CH, CH)], yv[b], semy[b]
            )

    def work(t, b):
        issue(t + 1, 1 - b)

        @pl.when(t < cnt)
        def _():
            pltpu.make_async_copy(
                x1_hbm.at[idxv.at[pl.ds(t * CE, CE)]], qg[b], sem[b]
            ).wait()
            pltpu.make_async_copy(
                y_hbm.at[pl.ds((base + t) * CH, CH)], yv[b], semy[b]
            ).wait()

            @pl.when(t >= 2)
            def _():
                ocid = base + t - 2
                pltpu.make_async_copy(
                    vb[b], vals_hbm.at[pl.ds(ocid * CH, CH)], semo[b]
                ).wait()
                pltpu.make_async_copy(
                    cb[b], sel_hbm.at[pl.ds(ocid * CH, CH)], semo[b]
                ).wait()

            qg_v = qg[b]
            y_v = yv[b]
            vb_v = vb[b]
            cb_v = cb[b]

            bc = bc_v[...]
            lanes = lax.iota(_i32, 16)
            topmask = lanes < KK

            def row(r, _):
                yr = [y_v[r, pl.ds(16 * c, 16)] for c in range(DC)]

                for l in range(DEG):
                    e = r * DEG + l
                    p = yr[0] * qg_v[e, pl.ds(0, 16)]
                    for c in range(1, DC):
                        p = p + yr[c] * qg_v[e, pl.ds(16 * c, 16)]
                    pb_v[l, :] = p

                s = plsc.load_gather(pb_v, [lanes, jnp.full((16,), 0, _i32)])
                for c2 in range(1, DC):
                    s = s + plsc.load_gather(
                        pb_v, [lanes, jnp.full((16,), c2, _i32)]
                    )
                u = s + bc
                sig = 1.0 / (1.0 + jnp.exp(-u))

                colsv = idxv[pl.ds(t * CE + r * DEG, 16)]
                sk, sv = plsc.sort_key_val(sig, colsv, descending=True)
                e_ = jnp.where(topmask, jnp.exp(sk), 0.0)
                vb_v[r, :] = e_ / jnp.sum(e_)
                cb_v[r, :] = sv
                return 0

            lax.fori_loop(0, CH, row, 0, unroll=2)
            pltpu.async_copy(vb_v, vals_hbm.at[pl.ds((base + t) * CH, CH)], semo[b])
            pltpu.async_copy(cb_v, sel_hbm.at[pl.ds((base + t) * CH, CH)], semo[b])

    issue(0, 0)

    def outer(tt, _):
        work(2 * tt, 0)
        work(2 * tt + 1, 1)
        return 0

    lax.fori_loop(0, TPW // 2, outer, 0)

    tl = cnt - 1
    for b in range(2):
        tb = tl - ((tl - b) % 2)
        ocid = base + tb
        pltpu.make_async_copy(
            vb[b], vals_hbm.at[pl.ds(ocid * CH, CH)], semo[b]
        ).wait()
        pltpu.make_async_copy(
            cb[b], sel_hbm.at[pl.ds(ocid * CH, CH)], semo[b]
        ).wait()


def _sc_score(X1, Y, cols, bconst):
    return pl.kernel(
        _sc_score_body,
        out_type=[
            jax.ShapeDtypeStruct((N, 16), _f32),
            jax.ShapeDtypeStruct((N, 16), _i32),
        ],
        mesh=plsc.VectorSubcoreMesh(core_axis_name="c", subcore_axis_name="s"),
        compiler_params=pltpu.CompilerParams(needs_layout_passes=False),
        scratch_types=[
            pltpu.VMEM((TPW * CE,), _i32),
            pltpu.VMEM((CE, D), _f32),
            pltpu.VMEM((CE, D), _f32),
            pltpu.VMEM((CH, D), _f32),
            pltpu.VMEM((CH, D), _f32),
            pltpu.VMEM((16,), _f32),
            pltpu.VMEM((16, 16), _f32),
            pltpu.VMEM((CH, 16), _f32),
            pltpu.VMEM((CH, 16), _f32),
            pltpu.VMEM((CH, 16), _i32),
            pltpu.VMEM((CH, 16), _i32),
            pltpu.SemaphoreType.DMA,
            pltpu.SemaphoreType.DMA,
            pltpu.SemaphoreType.DMA,
            pltpu.SemaphoreType.DMA,
            pltpu.SemaphoreType.DMA,
            pltpu.SemaphoreType.DMA,
        ],
    )(X1, Y, cols, bconst)


def kernel(X, edge_cols, edge_vals, W_fc, b_fc, W_bil, b_bil, prelu_a):
    del prelu_a
    cols = edge_cols.astype(_i32)
    bfc_row = b_fc.reshape(1, D)
    bconst = jnp.full((16,), b_bil, _f32)

    X1 = _tc_x1(X, W_fc, bfc_row)
    agg = _sc_agg(X, cols, edge_vals)
    Y = _tc_x2y(agg, X, W_bil)
    vals16, sel16 = _sc_score(X1, Y, cols, bconst)
    return vals16[:, :KK], sel16[:, :KK]


# speedup vs baseline: 1.1345x; 1.1345x over previous
import jax
import jax.numpy as jnp
from jax import lax
from jax.experimental import pallas as pl
from jax.experimental.pallas import tpu as pltpu
from jax.experimental.pallas import tpu_sc as plsc

N = 10000
DEG = 16
D = 256
KK = 8
NC = 2
NS = 16
NW = NC * NS
CH = 8
CE = CH * DEG
NCHUNK = N // CH
CBASE = NCHUNK // NW
CEXTRA = NCHUNK % NW
TPW = CBASE + 1
DC = D // 16
TCB = 1000

_f32 = jnp.float32
_i32 = jnp.int32


def _x1_body(wfc, bfc, x, x1):
    x1[...] = jnp.dot(x[...], wfc[...], preferred_element_type=_f32) + bfc[...]


def _tc_x1(X, W_fc, bfc_row):
    return pl.pallas_call(
        _x1_body,
        grid=(N // TCB,),
        in_specs=[
            pl.BlockSpec((D, D), lambda i: (0, 0)),
            pl.BlockSpec((1, D), lambda i: (0, 0)),
            pl.BlockSpec((TCB, D), lambda i: (i, 0)),
        ],
        out_specs=pl.BlockSpec((TCB, D), lambda i: (i, 0)),
        out_shape=jax.ShapeDtypeStruct((N, D), _f32),
    )(W_fc, bfc_row, X)


def _x2y_body(agg, x, wbil, y):
    h = agg[...] + x[...]
    n = jnp.sqrt(jnp.sum(h * h, axis=1, keepdims=True))
    x2 = h / jnp.maximum(n, 1e-12)
    y[...] = jnp.dot(x2, wbil[...], preferred_element_type=_f32)


def _tc_x2y(agg, X, W_bil):
    return pl.pallas_call(
        _x2y_body,
        grid=(N // TCB,),
        in_specs=[
            pl.BlockSpec((TCB, D), lambda i: (i, 0)),
            pl.BlockSpec((TCB, D), lambda i: (i, 0)),
            pl.BlockSpec((D, D), lambda i: (0, 0)),
        ],
        out_specs=pl.BlockSpec((TCB, D), lambda i: (i, 0)),
        out_shape=jax.ShapeDtypeStruct((N, D), _f32),
    )(agg, X, W_bil)


def _worker_extent(wid):
    cnt = CBASE + (wid < CEXTRA).astype(_i32)
    base = CBASE * wid + jnp.minimum(wid, CEXTRA)
    return base, cnt


def _stage_edges(src_hbm, dst_v, e0w):
    pltpu.sync_copy(src_hbm.at[pl.ds(e0w, CBASE * CE)], dst_v.at[pl.ds(0, CBASE * CE)])


def _stage_edges_tail(src_hbm, dst_v, e0w, cnt):
    @pl.when(cnt > CBASE)
    def _():
        pltpu.sync_copy(
            src_hbm.at[pl.ds(e0w + CBASE * CE, CE)],
            dst_v.at[pl.ds(CBASE * CE, CE)],
        )


def _sc_agg_body(
    x_hbm, cols_hbm, vals_hbm, agg_hbm,
    idxv, vvv, xg0, xg1, ab0, ab1, sem0, sem1, semo0, semo1,
):
    wid = lax.axis_index("s") * NC + lax.axis_index("c")
    base, cnt = _worker_extent(wid)
    e0w = base * CE
    _stage_edges(cols_hbm, idxv, e0w)
    _stage_edges(vals_hbm, vvv, e0w)
    _stage_edges_tail(cols_hbm, idxv, e0w, cnt)
    _stage_edges_tail(vals_hbm, vvv, e0w, cnt)
    xg = (xg0, xg1)
    ab = (ab0, ab1)
    sem = (sem0, sem1)
    semo = (semo0, semo1)

    def gather(t, b):
        return pltpu.make_async_copy(
            x_hbm.at[idxv.at[pl.ds(t * CE, CE)]], xg[b], sem[b]
        )

    def issue(t, b):
        @pl.when(t < cnt)
        def _():
            pltpu.async_copy(x_hbm.at[idxv.at[pl.ds(t * CE, CE)]], xg[b], sem[b])

    def work(t, b):
        issue(t + 1, 1 - b)

        @pl.when(t < cnt)
        def _():
            gather(t, b).wait()

            @pl.when(t >= 2)
            def _():
                pltpu.make_async_copy(
                    ab[b], agg_hbm.at[pl.ds((base + t - 2) * CH, CH)], semo[b]
                ).wait()

            xg_v = xg[b]
            ab_v = ab[b]

            def row(r, _):
                eb = t * CE + r * DEG
                accs = [jnp.zeros((16,), _f32) for _ in range(DC)]
                for l in range(DEG):
                    w = plsc.load_gather(vvv, [jnp.full((16,), eb + l, _i32)])
                    eg = r * DEG + l
                    for c in range(DC):
                        accs[c] = accs[c] + w * xg_v[eg, pl.ds(16 * c, 16)]
                for c in range(DC):
                    ab_v[r, pl.ds(16 * c, 16)] = accs[c]
                return 0

            lax.fori_loop(0, CH, row, 0)
            pltpu.async_copy(ab_v, agg_hbm.at[pl.ds((base + t) * CH, CH)], semo[b])

    issue(0, 0)

    def outer(tt, _):
        work(2 * tt, 0)
        work(2 * tt + 1, 1)
        return 0

    lax.fori_loop(0, TPW // 2, outer, 0)

    tl = cnt - 1
    for b in range(2):
        tb = tl - ((tl - b) % 2)
        pltpu.make_async_copy(
            ab[b], agg_hbm.at[pl.ds((base + tb) * CH, CH)], semo[b]
        ).wait()


def _sc_agg(X, cols, vals):
    return pl.kernel(
        _sc_agg_body,
        out_type=jax.ShapeDtypeStruct((N, D), _f32),
        mesh=plsc.VectorSubcoreMesh(core_axis_name="c", subcore_axis_name="s"),
        compiler_params=pltpu.CompilerParams(needs_layout_passes=False),
        scratch_types=[
            pltpu.VMEM((TPW * CE,), _i32),
            pltpu.VMEM((TPW * CE,), _f32),
            pltpu.VMEM((CE, D), _f32),
            pltpu.VMEM((CE, D), _f32),
            pltpu.VMEM((CH, D), _f32),
            pltpu.VMEM((CH, D), _f32),
            pltpu.SemaphoreType.DMA,
            pltpu.SemaphoreType.DMA,
            pltpu.SemaphoreType.DMA,
            pltpu.SemaphoreType.DMA,
        ],
    )(X, cols, vals)


def _sc_score_body(
    x1_hbm, y_hbm, cols_hbm, bc_hbm,
    vals_hbm, sel_hbm,
    idxv, qg0, qg1, y0, y1, bc_v, pb_v,
    vb0, vb1, cb0, cb1, sem0, sem1, semy0, semy1, semo0, semo1,
):
    wid = lax.axis_index("s") * NC + lax.axis_index("c")
    base, cnt = _worker_extent(wid)
    e0w = base * CE
    pltpu.sync_copy(bc_hbm, bc_v)
    _stage_edges(cols_hbm, idxv, e0w)
    _stage_edges_tail(cols_hbm, idxv, e0w, cnt)
    qg = (qg0, qg1)
    yv = (y0, y1)
    vb = (vb0, vb1)
    cb = (cb0, cb1)
    sem = (sem0, sem1)
    semy = (semy0, semy1)
    semo = (semo0, semo1)

    def issue(t, b):
        @pl.when(t < cnt)
        def _():
            pltpu.async_copy(x1_hbm.at[idxv.at[pl.ds(t * CE, CE)]], qg[b], sem[b])
            pltpu.async_copy(
                y_hbm.at[pl.ds((base + t) * CH, CH)], yv[b], semy[b]
            )

    def work(t, b):
        issue(t + 1, 1 - b)

        @pl.when(t < cnt)
        def _():
            pltpu.make_async_copy(
                x1_hbm.at[idxv.at[pl.ds(t * CE, CE)]], qg[b], sem[b]
            ).wait()
            pltpu.make_async_copy(
                y_hbm.at[pl.ds((base + t) * CH, CH)], yv[b], semy[b]
            ).wait()

            @pl.when(t >= 2)
            def _():
                ocid = base + t - 2
                pltpu.make_async_copy(
                    vb[b], vals_hbm.at[pl.ds(ocid * CH, CH)], semo[b]
                ).wait()
                pltpu.make_async_copy(
                    cb[b], sel_hbm.at[pl.ds(ocid * CH, CH)], semo[b]
                ).wait()

            qg_v = qg[b]
            y_v = yv[b]
            vb_v = vb[b]
            cb_v = cb[b]

            bc = bc_v[...]
            lanes = lax.iota(_i32, 16)
            topmask = lanes < KK

            def row(r, _):
                yr = [y_v[r, pl.ds(16 * c, 16)] for c in range(DC)]

                for l in range(DEG):
                    e = r * DEG + l
                    p = yr[0] * qg_v[e, pl.ds(0, 16)]
                    for c in range(1, DC):
                        p = p + yr[c] * qg_v[e, pl.ds(16 * c, 16)]
                    pb_v[l, :] = p

                s = plsc.load_gather(pb_v, [lanes, jnp.full((16,), 0, _i32)])
                for c2 in range(1, DC):
                    s = s + plsc.load_gather(
                        pb_v, [lanes, jnp.full((16,), c2, _i32)]
                    )
                u = s + bc
                sig = 1.0 / (1.0 + jnp.exp(-u))

                colsv = idxv[pl.ds(t * CE + r * DEG, 16)]
                sk, sv = plsc.sort_key_val(sig, colsv, descending=True)
                e_ = jnp.where(topmask, jnp.exp(sk), 0.0)
                vb_v[r, :] = e_ / jnp.sum(e_)
                cb_v[r, :] = sv
                return 0

            lax.fori_loop(0, CH, row, 0)
            pltpu.async_copy(vb_v, vals_hbm.at[pl.ds((base + t) * CH, CH)], semo[b])
            pltpu.async_copy(cb_v, sel_hbm.at[pl.ds((base + t) * CH, CH)], semo[b])

    issue(0, 0)

    def outer(tt, _):
        work(2 * tt, 0)
        work(2 * tt + 1, 1)
        return 0

    lax.fori_loop(0, TPW // 2, outer, 0)

    tl = cnt - 1
    for b in range(2):
        tb = tl - ((tl - b) % 2)
        ocid = base + tb
        pltpu.make_async_copy(
            vb[b], vals_hbm.at[pl.ds(ocid * CH, CH)], semo[b]
        ).wait()
        pltpu.make_async_copy(
            cb[b], sel_hbm.at[pl.ds(ocid * CH, CH)], semo[b]
        ).wait()


def _sc_score(X1, Y, cols, bconst):
    return pl.kernel(
        _sc_score_body,
        out_type=[
            jax.ShapeDtypeStruct((N, 16), _f32),
            jax.ShapeDtypeStruct((N, 16), _i32),
        ],
        mesh=plsc.VectorSubcoreMesh(core_axis_name="c", subcore_axis_name="s"),
        compiler_params=pltpu.CompilerParams(needs_layout_passes=False),
        scratch_types=[
            pltpu.VMEM((TPW * CE,), _i32),
            pltpu.VMEM((CE, D), _f32),
            pltpu.VMEM((CE, D), _f32),
            pltpu.VMEM((CH, D), _f32),
            pltpu.VMEM((CH, D), _f32),
            pltpu.VMEM((16,), _f32),
            pltpu.VMEM((16, 16), _f32),
            pltpu.VMEM((CH, 16), _f32),
            pltpu.VMEM((CH, 16), _f32),
            pltpu.VMEM((CH, 16), _i32),
            pltpu.VMEM((CH, 16), _i32),
            pltpu.SemaphoreType.DMA,
            pltpu.SemaphoreType.DMA,
            pltpu.SemaphoreType.DMA,
            pltpu.SemaphoreType.DMA,
            pltpu.SemaphoreType.DMA,
            pltpu.SemaphoreType.DMA,
        ],
    )(X1, Y, cols, bconst)


def kernel(X, edge_cols, edge_vals, W_fc, b_fc, W_bil, b_bil, prelu_a):
    del prelu_a
    cols = edge_cols.astype(_i32)
    bfc_row = b_fc.reshape(1, D)
    bconst = jnp.full((16,), b_bil, _f32)

    X1 = _tc_x1(X, W_fc, bfc_row)
    agg = _sc_agg(X, cols, edge_vals)
    Y = _tc_x2y(agg, X, W_bil)
    vals16, sel16 = _sc_score(X1, Y, cols, bconst)
    return vals16[:, :KK], sel16[:, :KK]


# speedup vs baseline: 1.2162x; 1.0721x over previous
import jax
import jax.numpy as jnp
from jax import lax
from jax.experimental import pallas as pl
from jax.experimental.pallas import tpu as pltpu
from jax.experimental.pallas import tpu_sc as plsc

N = 10000
DEG = 16
D = 256
KK = 8
NC = 2
NS = 16
NW = NC * NS
CH = 8
CE = CH * DEG
NCHUNK = N // CH
CBASE = NCHUNK // NW
CEXTRA = NCHUNK % NW
TPW = CBASE + 1
DC = D // 16
TCB = 1000

_f32 = jnp.float32
_i32 = jnp.int32


def _dense_body(wfc, bfc, wbil, agg, x, x1, y):
    xb = x[...]
    x1[...] = jnp.dot(xb, wfc[...], preferred_element_type=_f32) + bfc[...]
    h = agg[...] + xb
    n = jnp.sqrt(jnp.sum(h * h, axis=1, keepdims=True))
    x2 = h / jnp.maximum(n, 1e-12)
    y[...] = jnp.dot(x2, wbil[...], preferred_element_type=_f32)


def _tc_dense(agg, X, W_fc, bfc_row, W_bil):
    return pl.pallas_call(
        _dense_body,
        grid=(N // TCB,),
        in_specs=[
            pl.BlockSpec((D, D), lambda i: (0, 0)),
            pl.BlockSpec((1, D), lambda i: (0, 0)),
            pl.BlockSpec((D, D), lambda i: (0, 0)),
            pl.BlockSpec((TCB, D), lambda i: (i, 0)),
            pl.BlockSpec((TCB, D), lambda i: (i, 0)),
        ],
        out_specs=[
            pl.BlockSpec((TCB, D), lambda i: (i, 0)),
            pl.BlockSpec((TCB, D), lambda i: (i, 0)),
        ],
        out_shape=[
            jax.ShapeDtypeStruct((N, D), _f32),
            jax.ShapeDtypeStruct((N, D), _f32),
        ],
    )(W_fc, bfc_row, W_bil, agg, X)


def _worker_extent(wid):
    cnt = CBASE + (wid < CEXTRA).astype(_i32)
    base = CBASE * wid + jnp.minimum(wid, CEXTRA)
    return base, cnt


def _stage_edges(src_hbm, dst_v, e0w):
    pltpu.sync_copy(src_hbm.at[pl.ds(e0w, CBASE * CE)], dst_v.at[pl.ds(0, CBASE * CE)])


def _stage_edges_tail(src_hbm, dst_v, e0w, cnt):
    @pl.when(cnt > CBASE)
    def _():
        pltpu.sync_copy(
            src_hbm.at[pl.ds(e0w + CBASE * CE, CE)],
            dst_v.at[pl.ds(CBASE * CE, CE)],
        )


def _sc_agg_body(
    x_hbm, cols_hbm, vals_hbm, agg_hbm,
    idxv, vvv, xg0, xg1, ab0, ab1, sem0, sem1, semo0, semo1,
):
    wid = lax.axis_index("s") * NC + lax.axis_index("c")
    base, cnt = _worker_extent(wid)
    e0w = base * CE
    _stage_edges(cols_hbm, idxv, e0w)
    _stage_edges(vals_hbm, vvv, e0w)
    _stage_edges_tail(cols_hbm, idxv, e0w, cnt)
    _stage_edges_tail(vals_hbm, vvv, e0w, cnt)
    xg = (xg0, xg1)
    ab = (ab0, ab1)
    sem = (sem0, sem1)
    semo = (semo0, semo1)

    def gather(t, b):
        return pltpu.make_async_copy(
            x_hbm.at[idxv.at[pl.ds(t * CE, CE)]], xg[b], sem[b]
        )

    def issue(t, b):
        @pl.when(t < cnt)
        def _():
            pltpu.async_copy(x_hbm.at[idxv.at[pl.ds(t * CE, CE)]], xg[b], sem[b])

    def work(t, b):
        issue(t + 1, 1 - b)

        @pl.when(t < cnt)
        def _():
            gather(t, b).wait()

            @pl.when(t >= 2)
            def _():
                pltpu.make_async_copy(
                    ab[b], agg_hbm.at[pl.ds((base + t - 2) * CH, CH)], semo[b]
                ).wait()

            xg_v = xg[b]
            ab_v = ab[b]

            def row(r, _):
                eb = t * CE + r * DEG
                accs = [jnp.zeros((16,), _f32) for _ in range(DC)]
                for l in range(DEG):
                    w = plsc.load_gather(vvv, [jnp.full((16,), eb + l, _i32)])
                    eg = r * DEG + l
                    for c in range(DC):
                        accs[c] = accs[c] + w * xg_v[eg, pl.ds(16 * c, 16)]
                for c in range(DC):
                    ab_v[r, pl.ds(16 * c, 16)] = accs[c]
                return 0

            lax.fori_loop(0, CH, row, 0)
            pltpu.async_copy(ab_v, agg_hbm.at[pl.ds((base + t) * CH, CH)], semo[b])

    issue(0, 0)

    def outer(tt, _):
        work(2 * tt, 0)
        work(2 * tt + 1, 1)
        return 0

    lax.fori_loop(0, TPW // 2, outer, 0)

    tl = cnt - 1
    for b in range(2):
        tb = tl - ((tl - b) % 2)
        pltpu.make_async_copy(
            ab[b], agg_hbm.at[pl.ds((base + tb) * CH, CH)], semo[b]
        ).wait()


def _sc_agg(X, cols, vals):
    return pl.kernel(
        _sc_agg_body,
        out_type=jax.ShapeDtypeStruct((N, D), _f32),
        mesh=plsc.VectorSubcoreMesh(core_axis_name="c", subcore_axis_name="s"),
        compiler_params=pltpu.CompilerParams(needs_layout_passes=False),
        scratch_types=[
            pltpu.VMEM((TPW * CE,), _i32),
            pltpu.VMEM((TPW * CE,), _f32),
            pltpu.VMEM((CE, D), _f32),
            pltpu.VMEM((CE, D), _f32),
            pltpu.VMEM((CH, D), _f32),
            pltpu.VMEM((CH, D), _f32),
            pltpu.SemaphoreType.DMA,
            pltpu.SemaphoreType.DMA,
            pltpu.SemaphoreType.DMA,
            pltpu.SemaphoreType.DMA,
        ],
    )(X, cols, vals)


def _sc_score_body(
    x1_hbm, y_hbm, cols_hbm, bc_hbm,
    vals_hbm, sel_hbm,
    idxv, qg0, qg1, y0, y1, bc_v, pb_v,
    vb0, vb1, cb0, cb1, sem0, sem1, semy0, semy1, semo0, semo1,
):
    wid = lax.axis_index("s") * NC + lax.axis_index("c")
    base, cnt = _worker_extent(wid)
    e0w = base * CE
    pltpu.sync_copy(bc_hbm, bc_v)
    _stage_edges(cols_hbm, idxv, e0w)
    _stage_edges_tail(cols_hbm, idxv, e0w, cnt)
    qg = (qg0, qg1)
    yv = (y0, y1)
    vb = (vb0, vb1)
    cb = (cb0, cb1)
    sem = (sem0, sem1)
    semy = (semy0, semy1)
    semo = (semo0, semo1)

    def issue(t, b):
        @pl.when(t < cnt)
        def _():
            pltpu.async_copy(x1_hbm.at[idxv.at[pl.ds(t * CE, CE)]], qg[b], sem[b])
            pltpu.async_copy(
                y_hbm.at[pl.ds((base + t) * CH, CH)], yv[b], semy[b]
            )

    def work(t, b):
        issue(t + 1, 1 - b)

        @pl.when(t < cnt)
        def _():
            pltpu.make_async_copy(
                x1_hbm.at[idxv.at[pl.ds(t * CE, CE)]], qg[b], sem[b]
            ).wait()
            pltpu.make_async_copy(
                y_hbm.at[pl.ds((base + t) * CH, CH)], yv[b], semy[b]
            ).wait()

            @pl.when(t >= 2)
            def _():
                ocid = base + t - 2
                pltpu.make_async_copy(
                    vb[b].at[pl.ds(0, CH * KK)],
                    vals_hbm.at[pl.ds(ocid * CH * KK, CH * KK)],
                    semo[b],
                ).wait()
                pltpu.make_async_copy(
                    cb[b].at[pl.ds(0, CH * KK)],
                    sel_hbm.at[pl.ds(ocid * CH * KK, CH * KK)],
                    semo[b],
                ).wait()

            qg_v = qg[b]
            y_v = yv[b]
            vb_v = vb[b]
            cb_v = cb[b]

            bc = bc_v[...]
            lanes = lax.iota(_i32, 16)
            topmask = lanes < KK

            def row(r, _):
                yr = [y_v[r, pl.ds(16 * c, 16)] for c in range(DC)]

                for l in range(DEG):
                    e = r * DEG + l
                    acc = [yr[j] * qg_v[e, pl.ds(16 * j, 16)] for j in range(4)]
                    for c in range(4, DC, 4):
                        for j in range(4):
                            acc[j] = acc[j] + yr[c + j] * qg_v[e, pl.ds(16 * (c + j), 16)]
                    pb_v[l, :] = (acc[0] + acc[1]) + (acc[2] + acc[3])

                s = plsc.load_gather(pb_v, [lanes, jnp.full((16,), 0, _i32)])
                for c2 in range(1, DC):
                    s = s + plsc.load_gather(
                        pb_v, [lanes, jnp.full((16,), c2, _i32)]
                    )
                u = s + bc
                sig = 1.0 / (1.0 + jnp.exp(-u))

                colsv = idxv[pl.ds(t * CE + r * DEG, 16)]
                sk, sv = plsc.sort_key_val(sig, colsv, descending=True)
                e_ = jnp.where(topmask, jnp.exp(sk), 0.0)
                plsc.store_compressed(
                    vb_v.at[pl.ds(r * KK, 16)], e_ / jnp.sum(e_), mask=topmask
                )
                plsc.store_compressed(cb_v.at[pl.ds(r * KK, 16)], sv, mask=topmask)
                return 0

            lax.fori_loop(0, CH, row, 0)
            pltpu.async_copy(
                vb_v.at[pl.ds(0, CH * KK)],
                vals_hbm.at[pl.ds((base + t) * CH * KK, CH * KK)],
                semo[b],
            )
            pltpu.async_copy(
                cb_v.at[pl.ds(0, CH * KK)],
                sel_hbm.at[pl.ds((base + t) * CH * KK, CH * KK)],
                semo[b],
            )

    issue(0, 0)

    def outer(tt, _):
        work(2 * tt, 0)
        work(2 * tt + 1, 1)
        return 0

    lax.fori_loop(0, TPW // 2, outer, 0)

    tl = cnt - 1
    for b in range(2):
        tb = tl - ((tl - b) % 2)
        ocid = base + tb
        pltpu.make_async_copy(
            vb[b].at[pl.ds(0, CH * KK)],
            vals_hbm.at[pl.ds(ocid * CH * KK, CH * KK)],
            semo[b],
        ).wait()
        pltpu.make_async_copy(
            cb[b].at[pl.ds(0, CH * KK)],
            sel_hbm.at[pl.ds(ocid * CH * KK, CH * KK)],
            semo[b],
        ).wait()


def _sc_score(X1, Y, cols, bconst):
    return pl.kernel(
        _sc_score_body,
        out_type=[
            jax.ShapeDtypeStruct((N * KK,), _f32),
            jax.ShapeDtypeStruct((N * KK,), _i32),
        ],
        mesh=plsc.VectorSubcoreMesh(core_axis_name="c", subcore_axis_name="s"),
        compiler_params=pltpu.CompilerParams(needs_layout_passes=False),
        scratch_types=[
            pltpu.VMEM((TPW * CE,), _i32),
            pltpu.VMEM((CE, D), _f32),
            pltpu.VMEM((CE, D), _f32),
            pltpu.VMEM((CH, D), _f32),
            pltpu.VMEM((CH, D), _f32),
            pltpu.VMEM((16,), _f32),
            pltpu.VMEM((16, 16), _f32),
            pltpu.VMEM((CH * KK + 16,), _f32),
            pltpu.VMEM((CH * KK + 16,), _f32),
            pltpu.VMEM((CH * KK + 16,), _i32),
            pltpu.VMEM((CH * KK + 16,), _i32),
            pltpu.SemaphoreType.DMA,
            pltpu.SemaphoreType.DMA,
            pltpu.SemaphoreType.DMA,
            pltpu.SemaphoreType.DMA,
            pltpu.SemaphoreType.DMA,
            pltpu.SemaphoreType.DMA,
        ],
    )(X1, Y, cols, bconst)


def kernel(X, edge_cols, edge_vals, W_fc, b_fc, W_bil, b_bil, prelu_a):
    del prelu_a
    cols = edge_cols.astype(_i32)
    bfc_row = b_fc.reshape(1, D)
    bconst = jnp.full((16,), b_bil, _f32)

    agg = _sc_agg(X, cols, edge_vals)
    X1, Y = _tc_dense(agg, X, W_fc, bfc_row, W_bil)
    vals_flat, sel_flat = _sc_score(X1, Y, cols, bconst)
    return vals_flat.reshape(N, KK), sel_flat.reshape(N, KK)
